# baseline (speedup 1.0000x reference)
import jax
import jax.numpy as jnp
from jax.experimental import pallas as pl

NUM_USERS = 50000
NUM_ITEMS = 50000
PRUNE_THRESHOLD = 0.02
N_LAYERS = 3


def kernel(inter_user, inter_item, inter_data, Wu, Wi):
    eps = 1e-8
    user_emb = jax.ops.segment_sum(inter_data[:, None] * jnp.take(Wu.T, inter_item, axis=0), inter_user, num_segments=NUM_USERS)
    item_emb = jax.ops.segment_sum(inter_data[:, None] * jnp.take(Wi.T, inter_user, axis=0), inter_item, num_segments=NUM_ITEMS)
    u_n = jnp.linalg.norm(user_emb, axis=1, keepdims=True)
    i_n = jnp.linalg.norm(item_emb, axis=1, keepdims=True)
    u_norm = user_emb / jnp.maximum(u_n, eps)
    i_norm = item_emb / jnp.maximum(i_n, eps)
    sims = jnp.sum(jnp.take(u_norm, inter_user, axis=0) * jnp.take(i_norm, inter_item, axis=0), axis=1)
    row = jnp.concatenate([inter_user, inter_item + NUM_USERS])
    col = jnp.concatenate([inter_item + NUM_USERS, inter_user])
    sim_value = (jnp.concatenate([sims, sims]) + 1.0) / 2.0
    pruned = jnp.where(sim_value < PRUNE_THRESHOLD, jnp.zeros_like(sim_value), sim_value)
    N = NUM_USERS + NUM_ITEMS
    diags = jax.ops.segment_sum(pruned, row, num_segments=N) + 1e-7
    d_inv = 1.0 / diags
    normal_val = pruned * jnp.take(d_inv, row, axis=0)
    all_emb = jnp.concatenate([user_emb, item_emb], axis=0)
    embs = [all_emb]
    for _ in range(N_LAYERS):
        all_emb = jax.ops.segment_sum(normal_val[:, None] * jnp.take(all_emb, col, axis=0), row, num_segments=N)
        embs.append(all_emb)
    light_out = jnp.mean(jnp.stack(embs, axis=1), axis=1)
    return light_out
